# Initial kernel scaffold; baseline (speedup 1.0000x reference)
import jax
import jax.numpy as jnp
from jax.experimental import pallas as pl


def kernel(edge, edge_w, N, E, out_features):
    raise NotImplementedError("write your pallas kernel here")


# speedup vs baseline: 4.4104x; 4.4104x over previous
import jax
import jax.numpy as jnp
from jax import lax
from jax.experimental import pallas as pl
from jax.experimental.pallas import tpu as pltpu
from jax.experimental.pallas import tpu_sc as plsc

_N = 10000
_E = 320000
_D = 128

_info = plsc.get_sparse_core_info()
_NC = _info.num_cores
_NS = _info.num_subcores
_L = _info.num_lanes

_EPC = _E // _NC
_EPT = _EPC // _NS
_CH = 128
_NFULL = _EPT // _CH
_TAIL = _EPT - _NFULL * _CH
_RPT = 632
_RLAST = _N - (_NS - 1) * _RPT
_ZR = 8


def _body(edge_hbm, ew_hbm, out_hbm, acc, ibuf, wbuf, ibuf_t, wbuf_t, zbuf):
    c = lax.axis_index("c")
    s = lax.axis_index("s")
    base = c * _EPC + s * _EPT

    zeros = jnp.zeros((_L,), jnp.float32)
    for r in range(_ZR):
        for j in range(_D // _L):
            zbuf[r, pl.ds(j * _L, _L)] = zeros

    nrows = jnp.where(s == _NS - 1, _RLAST, _RPT)
    r0 = s * _RPT

    def zblk(t, carry):
        pltpu.sync_copy(zbuf, acc.at[pl.ds(r0 + t * _ZR, _ZR)])
        return carry

    lax.fori_loop(0, nrows // _ZR, zblk, 0)
    plsc.subcore_barrier()

    def chunk(k, carry):
        off = base + k * _CH
        pltpu.sync_copy(edge_hbm.at[pl.ds(off, _CH)], ibuf.at[0])
        pltpu.sync_copy(ew_hbm.at[pl.ds(off, _CH)], wbuf)
        pltpu.sync_copy(wbuf, acc.at[ibuf.at[0]], add=True)
        return carry

    lax.fori_loop(0, _NFULL, chunk, 0)

    off = base + _NFULL * _CH
    pltpu.sync_copy(edge_hbm.at[pl.ds(off, _TAIL)], ibuf_t.at[0])
    pltpu.sync_copy(ew_hbm.at[pl.ds(off, _TAIL)], wbuf_t)
    pltpu.sync_copy(wbuf_t, acc.at[ibuf_t.at[0]], add=True)

    plsc.subcore_barrier()
    pltpu.sync_copy(acc.at[pl.ds(r0, nrows)], out_hbm.at[c, pl.ds(r0, nrows)])


_scatter = pl.kernel(
    _body,
    out_type=jax.ShapeDtypeStruct((_NC, _N, _D), jnp.float32),
    mesh=plsc.VectorSubcoreMesh(core_axis_name="c", subcore_axis_name="s"),
    scratch_types=[
        pltpu.VMEM_SHARED((_N, _D), jnp.float32),
        pltpu.VMEM((1, _CH), jnp.int32),
        pltpu.VMEM((_CH, _D), jnp.float32),
        pltpu.VMEM((1, _TAIL), jnp.int32),
        pltpu.VMEM((_TAIL, _D), jnp.float32),
        pltpu.VMEM((_ZR, _D), jnp.float32),
    ],
)


def _combine_body(p_ref, o_ref):
    o_ref[...] = p_ref[0] + p_ref[1]


def _combine(partials):
    grid = 10
    rows = _N // grid
    return pl.pallas_call(
        _combine_body,
        out_shape=jax.ShapeDtypeStruct((_N, _D), jnp.float32),
        grid=(grid,),
        in_specs=[pl.BlockSpec((_NC, rows, _D), lambda i: (0, i, 0))],
        out_specs=pl.BlockSpec((rows, _D), lambda i: (i, 0)),
    )(partials)


def kernel(edge, edge_w, N, E, out_features):
    edge0 = jnp.asarray(edge[0], jnp.int32)
    partials = _scatter(edge0, jnp.asarray(edge_w, jnp.float32))
    return _combine(partials)


# speedup vs baseline: 7.6684x; 1.7387x over previous
import jax
import jax.numpy as jnp
from jax import lax
from jax.experimental import pallas as pl
from jax.experimental.pallas import tpu as pltpu
from jax.experimental.pallas import tpu_sc as plsc

_N = 10000
_E = 320000
_D = 128

_info = plsc.get_sparse_core_info()
_NC = _info.num_cores
_NS = _info.num_subcores
_L = _info.num_lanes

_EPC = _E // _NC
_CH = 128
_EPTA = 10112
_CHA = _EPTA // _CH
_CHB = (_EPC - (_NS - 1) * _EPTA) // _CH
_RPT = 632
_RLAST = _N - (_NS - 1) * _RPT
_ZR = 8


def _body(edge_hbm, ew_hbm, out_hbm, acc, ib0, ib1, wb0, wb1, zbuf,
          sem0, sem1):
    c = lax.axis_index("c")
    s = lax.axis_index("s")
    ebase = c * _EPC + s * _EPTA
    nchunks = jnp.where(s == _NS - 1, _CHB, _CHA)
    ibufs = (ib0, ib1)
    wbufs = (wb0, wb1)
    sems = (sem0, sem1)

    def start_load(k, p):
        off = ebase + k * _CH
        pltpu.async_copy(edge_hbm.at[pl.ds(off, _CH)], ibufs[p].at[0], sems[p])
        pltpu.async_copy(ew_hbm.at[pl.ds(off, _CH)], wbufs[p], sems[p])

    def wait_load(k, p):
        off = ebase + k * _CH
        pltpu.make_async_copy(
            edge_hbm.at[pl.ds(off, _CH)], ibufs[p].at[0], sems[p]).wait()
        pltpu.make_async_copy(
            ew_hbm.at[pl.ds(off, _CH)], wbufs[p], sems[p]).wait()

    def scatter(p):
        pltpu.sync_copy(wbufs[p], acc.at[ibufs[p].at[0]], add=True)

    start_load(0, 0)

    zeros = jnp.zeros((_L,), jnp.float32)
    for r in range(_ZR):
        for j in range(_D // _L):
            zbuf[r, pl.ds(j * _L, _L)] = zeros

    nrows = jnp.where(s == _NS - 1, _RLAST, _RPT)
    r0 = s * _RPT

    def zblk(t, carry):
        pltpu.sync_copy(zbuf, acc.at[pl.ds(r0 + t * _ZR, _ZR)])
        return carry

    lax.fori_loop(0, nrows // _ZR, zblk, 0)
    plsc.subcore_barrier()

    def group(g, carry):
        k0 = 2 * g
        start_load(k0 + 1, 1)
        wait_load(k0, 0)
        scatter(0)
        start_load(k0 + 2, 0)
        wait_load(k0 + 1, 1)
        scatter(1)
        return carry

    ngroups = nchunks // 2
    lax.fori_loop(0, ngroups, group, 0)
    last = 2 * ngroups
    wait_load(last, 0)
    scatter(0)

    plsc.subcore_barrier()
    pltpu.sync_copy(acc.at[pl.ds(r0, nrows)], out_hbm.at[c, pl.ds(r0, nrows)])


_scatter = pl.kernel(
    _body,
    out_type=jax.ShapeDtypeStruct((_NC, _N, _D), jnp.float32),
    mesh=plsc.VectorSubcoreMesh(core_axis_name="c", subcore_axis_name="s"),
    scratch_types=[
        pltpu.VMEM_SHARED((_N, _D), jnp.float32),
        pltpu.VMEM((1, _CH), jnp.int32),
        pltpu.VMEM((1, _CH), jnp.int32),
        pltpu.VMEM((_CH, _D), jnp.float32),
        pltpu.VMEM((_CH, _D), jnp.float32),
        pltpu.VMEM((_ZR, _D), jnp.float32),
        pltpu.SemaphoreType.DMA,
        pltpu.SemaphoreType.DMA,
    ],
)


def _combine_body(p_ref, o_ref):
    o_ref[...] = p_ref[0] + p_ref[1]


def _combine(partials):
    grid = 10
    rows = _N // grid
    return pl.pallas_call(
        _combine_body,
        out_shape=jax.ShapeDtypeStruct((_N, _D), jnp.float32),
        grid=(grid,),
        in_specs=[pl.BlockSpec((_NC, rows, _D), lambda i: (0, i, 0))],
        out_specs=pl.BlockSpec((rows, _D), lambda i: (i, 0)),
    )(partials)


def kernel(edge, edge_w, N, E, out_features):
    edge0 = jnp.asarray(edge[0], jnp.int32)
    partials = _scatter(edge0, jnp.asarray(edge_w, jnp.float32))
    return _combine(partials)


# speedup vs baseline: 8.6422x; 1.1270x over previous
import jax
import jax.numpy as jnp
from jax import lax
from jax.experimental import pallas as pl
from jax.experimental.pallas import tpu as pltpu
from jax.experimental.pallas import tpu_sc as plsc

_N = 10000
_E = 320000
_D = 128

_info = plsc.get_sparse_core_info()
_NC = _info.num_cores
_NS = _info.num_subcores
_L = _info.num_lanes

_EPC = _E // _NC
_CH = 128
_EPTA = 10112
_CHA = _EPTA // _CH
_CHB = (_EPC - (_NS - 1) * _EPTA) // _CH
_RPT = 632
_RLAST = _N - (_NS - 1) * _RPT
_ZR = 104


def _body(edge_hbm, ew_hbm, out_hbm, acc, ib0, ib1, wb0, wb1, zbuf,
          sem0, sem1):
    c = lax.axis_index("c")
    s = lax.axis_index("s")
    ebase = c * _EPC + s * _EPTA
    nchunks = jnp.where(s == _NS - 1, _CHB, _CHA)
    ibufs = (ib0, ib1)
    wbufs = (wb0, wb1)
    sems = (sem0, sem1)

    _H = _CH // 2

    def start_load(k, p):
        off = ebase + k * _CH
        pltpu.async_copy(
            edge_hbm.at[pl.ds(0, 1), pl.ds(off, _CH)], ibufs[p], sems[p])
        pltpu.async_copy(ew_hbm.at[pl.ds(off, _H)],
                         wbufs[p].at[pl.ds(0, _H)], sems[p])
        pltpu.async_copy(ew_hbm.at[pl.ds(off + _H, _H)],
                         wbufs[p].at[pl.ds(_H, _H)], sems[p])

    def wait_load(k, p):
        off = ebase + k * _CH
        pltpu.make_async_copy(
            edge_hbm.at[pl.ds(0, 1), pl.ds(off, _CH)], ibufs[p], sems[p]).wait()
        pltpu.make_async_copy(
            ew_hbm.at[pl.ds(off, _H)], wbufs[p].at[pl.ds(0, _H)],
            sems[p]).wait()
        pltpu.make_async_copy(
            ew_hbm.at[pl.ds(off + _H, _H)], wbufs[p].at[pl.ds(_H, _H)],
            sems[p]).wait()

    def scatter(p):
        pltpu.sync_copy(wbufs[p], acc.at[ibufs[p].at[0]], add=True)

    start_load(0, 0)

    zeros = jnp.zeros((_L,), jnp.float32)

    def zfill(r, carry):
        for j in range(_D // _L):
            zbuf[r, pl.ds(j * _L, _L)] = zeros
        return carry

    lax.fori_loop(0, _ZR, zfill, 0)

    nrows = jnp.where(s == _NS - 1, _RLAST, _RPT)
    r0 = s * _RPT

    def zblk(t, carry):
        pltpu.sync_copy(zbuf, acc.at[pl.ds(r0 + t * _ZR, _ZR)])
        return carry

    lax.fori_loop(0, nrows // _ZR, zblk, 0)
    @pl.when(s < _NS - 1)
    def _():
        pltpu.sync_copy(zbuf.at[pl.ds(0, _RPT - (_RPT // _ZR) * _ZR)],
                        acc.at[pl.ds(r0 + (_RPT // _ZR) * _ZR,
                                     _RPT - (_RPT // _ZR) * _ZR)])
    plsc.subcore_barrier()

    def group(g, carry):
        k0 = 2 * g
        start_load(k0 + 1, 1)
        wait_load(k0, 0)
        scatter(0)
        start_load(k0 + 2, 0)
        wait_load(k0 + 1, 1)
        scatter(1)
        return carry

    ngroups = nchunks // 2
    lax.fori_loop(0, ngroups, group, 0)
    last = 2 * ngroups
    wait_load(last, 0)
    scatter(0)

    plsc.subcore_barrier()
    pltpu.sync_copy(acc.at[pl.ds(r0, nrows)], out_hbm.at[c, pl.ds(r0, nrows)])


_scatter = pl.kernel(
    _body,
    out_type=jax.ShapeDtypeStruct((_NC, _N, _D), jnp.float32),
    mesh=plsc.VectorSubcoreMesh(core_axis_name="c", subcore_axis_name="s"),
    scratch_types=[
        pltpu.VMEM_SHARED((_N, _D), jnp.float32),
        pltpu.VMEM((1, _CH), jnp.int32),
        pltpu.VMEM((1, _CH), jnp.int32),
        pltpu.VMEM((_CH, _D), jnp.float32),
        pltpu.VMEM((_CH, _D), jnp.float32),
        pltpu.VMEM((_ZR, _D), jnp.float32),
        pltpu.SemaphoreType.DMA,
        pltpu.SemaphoreType.DMA,
    ],
)


def _combine_body(p_ref, o_ref):
    o_ref[...] = p_ref[0] + p_ref[1]


def _combine(partials):
    grid = 10
    rows = _N // grid
    return pl.pallas_call(
        _combine_body,
        out_shape=jax.ShapeDtypeStruct((_N, _D), jnp.float32),
        grid=(grid,),
        in_specs=[pl.BlockSpec((_NC, rows, _D), lambda i: (0, i, 0))],
        out_specs=pl.BlockSpec((rows, _D), lambda i: (i, 0)),
    )(partials)


def kernel(edge, edge_w, N, E, out_features):
    partials = _scatter(jnp.asarray(edge, jnp.int32),
                        jnp.asarray(edge_w, jnp.float32))
    return _combine(partials)


# speedup vs baseline: 8.8165x; 1.0202x over previous
import jax
import jax.numpy as jnp
from jax import lax
from jax.experimental import pallas as pl
from jax.experimental.pallas import tpu as pltpu
from jax.experimental.pallas import tpu_sc as plsc

_N = 10000
_E = 320000
_D = 128

_info = plsc.get_sparse_core_info()
_NC = _info.num_cores
_NS = _info.num_subcores
_L = _info.num_lanes

_EPC = _E // _NC
_CH = 128
_EPTA = 10112
_CHA = _EPTA // _CH
_CHB = (_EPC - (_NS - 1) * _EPTA) // _CH
_RPT = 632
_RLAST = _N - (_NS - 1) * _RPT
_ZR = 104


def _body(edge_hbm, ew_hbm, out_hbm, acc, ib0, ib1, wb0, wb1, zbuf,
          sem0, sem1):
    c = lax.axis_index("c")
    s = lax.axis_index("s")
    ebase = c * _EPC + s * _EPTA
    nchunks = jnp.where(s == _NS - 1, _CHB, _CHA)
    ibufs = (ib0, ib1)
    wbufs = (wb0, wb1)
    sems = (sem0, sem1)

    def start_load(k, p):
        off = ebase + k * _CH
        pltpu.async_copy(
            edge_hbm.at[pl.ds(0, 1), pl.ds(off, _CH)], ibufs[p], sems[p])
        pltpu.async_copy(ew_hbm.at[pl.ds(off, _CH)], wbufs[p], sems[p])

    def wait_load(k, p):
        off = ebase + k * _CH
        pltpu.make_async_copy(
            edge_hbm.at[pl.ds(0, 1), pl.ds(off, _CH)], ibufs[p], sems[p]).wait()
        pltpu.make_async_copy(
            ew_hbm.at[pl.ds(off, _CH)], wbufs[p], sems[p]).wait()

    def scatter(p):
        pltpu.sync_copy(wbufs[p], acc.at[ibufs[p].at[0]], add=True)

    start_load(0, 0)

    zeros = jnp.zeros((_L,), jnp.float32)

    def zfill(r, carry):
        for j in range(_D // _L):
            zbuf[r, pl.ds(j * _L, _L)] = zeros
        return carry

    lax.fori_loop(0, _ZR, zfill, 0)

    nrows = jnp.where(s == _NS - 1, _RLAST, _RPT)
    r0 = s * _RPT

    def zblk(t, carry):
        pltpu.sync_copy(zbuf, acc.at[pl.ds(r0 + t * _ZR, _ZR)])
        return carry

    lax.fori_loop(0, nrows // _ZR, zblk, 0)
    @pl.when(s < _NS - 1)
    def _():
        pltpu.sync_copy(zbuf.at[pl.ds(0, _RPT - (_RPT // _ZR) * _ZR)],
                        acc.at[pl.ds(r0 + (_RPT // _ZR) * _ZR,
                                     _RPT - (_RPT // _ZR) * _ZR)])
    plsc.subcore_barrier()

    def group(g, carry):
        k0 = 2 * g
        start_load(k0 + 1, 1)
        wait_load(k0, 0)
        scatter(0)
        start_load(k0 + 2, 0)
        wait_load(k0 + 1, 1)
        scatter(1)
        return carry

    ngroups = nchunks // 2
    lax.fori_loop(0, ngroups, group, 0)
    last = 2 * ngroups
    wait_load(last, 0)
    scatter(0)

    plsc.subcore_barrier()
    pltpu.sync_copy(acc.at[pl.ds(r0, nrows)], out_hbm.at[c, pl.ds(r0, nrows)])


_scatter = pl.kernel(
    _body,
    out_type=jax.ShapeDtypeStruct((_NC, _N, _D), jnp.float32),
    mesh=plsc.VectorSubcoreMesh(core_axis_name="c", subcore_axis_name="s"),
    scratch_types=[
        pltpu.VMEM_SHARED((_N, _D), jnp.float32),
        pltpu.VMEM((1, _CH), jnp.int32),
        pltpu.VMEM((1, _CH), jnp.int32),
        pltpu.VMEM((_CH, _D), jnp.float32),
        pltpu.VMEM((_CH, _D), jnp.float32),
        pltpu.VMEM((_ZR, _D), jnp.float32),
        pltpu.SemaphoreType.DMA,
        pltpu.SemaphoreType.DMA,
    ],
)


def _combine_body(p_ref, o_ref):
    o_ref[...] = p_ref[0] + p_ref[1]


def _combine(partials):
    grid = 5
    rows = _N // grid
    return pl.pallas_call(
        _combine_body,
        out_shape=jax.ShapeDtypeStruct((_N, _D), jnp.float32),
        grid=(grid,),
        in_specs=[pl.BlockSpec((_NC, rows, _D), lambda i: (0, i, 0))],
        out_specs=pl.BlockSpec((rows, _D), lambda i: (i, 0)),
    )(partials)


def kernel(edge, edge_w, N, E, out_features):
    partials = _scatter(jnp.asarray(edge, jnp.int32),
                        jnp.asarray(edge_w, jnp.float32))
    return _combine(partials)
